# baseline (speedup 1.0000x reference)
import functools

import jax
import jax.numpy as jnp
from jax import lax
from jax.experimental import pallas as pl
from jax.experimental.pallas import tpu as pltpu
from jax.experimental.pallas import tpu_sc as plsc

B = 4096
L1, L2, L3 = 20, 200, 26
H = 128
HW = H // 2
NHID = 256

NC, NS = 2, 16
NW = NC * NS
CH = 128

N1, N2, N3 = B * L1, B * L2, B * L3
P1, P2, P3 = N1 // NW, N2 // NW, N3 // NW


def _sc_gather():
    mesh = plsc.VectorSubcoreMesh(core_axis_name="c", subcore_axis_name="s")

    @functools.partial(
        pl.kernel,
        mesh=mesh,
        out_type=(
            jax.ShapeDtypeStruct((N1, HW), jnp.int32),
            jax.ShapeDtypeStruct((N2, HW), jnp.int32),
            jax.ShapeDtypeStruct((N3, HW), jnp.int32),
        ),
        scratch_types=[
            pltpu.VMEM((CH,), jnp.int32),
            pltpu.VMEM((CH, HW), jnp.int32),
            pltpu.SemaphoreType.DMA,
        ],
        compiler_params=pltpu.CompilerParams(use_tc_tiling_on_sc=False),
    )
    def k(idx1, idx2, idx3, t1, t2, t3, o1, o2, o3, idx_v, rows_v, sem):
        wid = lax.axis_index("s") * NC + lax.axis_index("c")

        def run(idx_hbm, table_hbm, out_hbm, per_worker):
            base = wid * per_worker

            def body(c, _):
                off = base + c * CH
                pltpu.sync_copy(idx_hbm.at[pl.ds(off, CH)], idx_v)
                pltpu.async_copy(table_hbm.at[idx_v], rows_v, sem).wait()
                pltpu.sync_copy(rows_v, out_hbm.at[pl.ds(off, CH)])
                return 0

            lax.fori_loop(0, per_worker // CH, body, 0)

        run(idx1, t1, o1, P1)
        run(idx2, t2, o2, P2)
        run(idx3, t3, o3, P3)

    return k


_MB = B
_KB = 256
_NK1, _NK2, _NK3 = (L1 * H) // _KB, (L2 * H) // _KB, (L3 * H) // _KB
_NK = _NK1 + _NK2 + _NK3


def _tc_body(g1, g2, g3, w1, w2, w3, bi, wf, bf, out_ref, acc_ref):
    k = pl.program_id(0)

    @pl.when(k == 0)
    def _():
        acc_ref[...] = jnp.broadcast_to(bi[...], (_MB, NHID))

    @pl.when(k < _NK1)
    def _():
        acc_ref[...] += jnp.dot(g1[...], w1[...], preferred_element_type=jnp.float32)

    @pl.when((k >= _NK1) & (k < _NK1 + _NK2))
    def _():
        acc_ref[...] += jnp.dot(g2[...], w2[...], preferred_element_type=jnp.float32)

    @pl.when(k >= _NK1 + _NK2)
    def _():
        acc_ref[...] += jnp.dot(g3[...], w3[...], preferred_element_type=jnp.float32)

    @pl.when(k == _NK - 1)
    def _():
        r = acc_ref[...] * wf[...]
        out_ref[...] = jnp.sum(r, axis=1, keepdims=True) + bf[0, 0]


def _tc_mlp(g1, g2, g3, w1, w2, w3, b_inter, w_final, b_final):
    grid = (_NK,)

    def seg_spec(lo, nk):
        return pl.BlockSpec(
            (_MB, _KB),
            lambda k: (0, jnp.clip(k - lo, 0, nk - 1)),
        )

    def w_spec(lo, nk):
        return pl.BlockSpec(
            (_KB, NHID),
            lambda k: (jnp.clip(k - lo, 0, nk - 1), 0),
        )

    return pl.pallas_call(
        _tc_body,
        grid=grid,
        in_specs=[
            seg_spec(0, _NK1),
            seg_spec(_NK1, _NK2),
            seg_spec(_NK1 + _NK2, _NK3),
            w_spec(0, _NK1),
            w_spec(_NK1, _NK2),
            w_spec(_NK1 + _NK2, _NK3),
            pl.BlockSpec((1, NHID), lambda k: (0, 0)),
            pl.BlockSpec((1, NHID), lambda k: (0, 0)),
            pl.BlockSpec(memory_space=pltpu.SMEM),
        ],
        out_specs=pl.BlockSpec((_MB, 1), lambda k: (0, 0)),
        out_shape=jax.ShapeDtypeStruct((B, 1), jnp.float32),
        scratch_shapes=[pltpu.VMEM((_MB, NHID), jnp.float32)],
    )(g1, g2, g3, w1, w2, w3, b_inter, w_final, b_final)


def _rearrange_w(w_seg, lt):
    return w_seg.reshape(NHID, H, lt).transpose(2, 1, 0).reshape(lt * H, NHID)


def _pack_table(t):
    return lax.bitcast_convert_type(
        t.astype(jnp.bfloat16).reshape(t.shape[0], HW, 2), jnp.int32)


def _unpack_rows(g, lt):
    return lax.bitcast_convert_type(g, jnp.bfloat16).reshape(B, lt * H)


def kernel(input1, input2, input3, title_emb, full_emb, cat_emb,
           W_inter, b_inter, W_final, b_final):
    idx1 = input1.reshape(-1).astype(jnp.int32)
    idx2 = input2.reshape(-1).astype(jnp.int32)
    idx3 = input3.reshape(-1).astype(jnp.int32)

    g1, g2, g3 = _sc_gather()(
        idx1, idx2, idx3,
        _pack_table(title_emb), _pack_table(full_emb), _pack_table(cat_emb))

    wb = W_inter.astype(jnp.bfloat16)
    w1 = _rearrange_w(wb[:, : H * L1], L1)
    w2 = _rearrange_w(wb[:, H * L1 : H * (L1 + L2)], L2)
    w3 = _rearrange_w(wb[:, H * (L1 + L2) :], L3)

    return _tc_mlp(
        _unpack_rows(g1, L1), _unpack_rows(g2, L2), _unpack_rows(g3, L3),
        w1, w2, w3,
        b_inter.reshape(1, NHID),
        W_final.reshape(1, NHID),
        b_final.reshape(1, 1).astype(jnp.float32),
    )


# speedup vs baseline: 36.8909x; 36.8909x over previous
import functools

import jax
import jax.numpy as jnp
from jax import lax
from jax.experimental import pallas as pl
from jax.experimental.pallas import tpu as pltpu
from jax.experimental.pallas import tpu_sc as plsc

B = 4096
L1, L2, L3 = 20, 200, 26
H = 128
HW = H // 2
NHID = 256

NC, NS = 2, 16
NW = NC * NS
CH = 128

N1, N2, N3 = B * L1, B * L2, B * L3
P1, P2, P3 = N1 // NW, N2 // NW, N3 // NW


def _sc_gather():
    mesh = plsc.VectorSubcoreMesh(core_axis_name="c", subcore_axis_name="s")

    @functools.partial(
        pl.kernel,
        mesh=mesh,
        out_type=(
            jax.ShapeDtypeStruct((N1, H), jnp.float32),
            jax.ShapeDtypeStruct((N2, H), jnp.float32),
            jax.ShapeDtypeStruct((N3, H), jnp.float32),
        ),
        scratch_types=[
            pltpu.VMEM((CH,), jnp.int32),
            pltpu.VMEM((CH, H), jnp.float32),
            pltpu.SemaphoreType.DMA,
        ],
    )
    def k(idx1, idx2, idx3, t1, t2, t3, o1, o2, o3, idx_v, rows_v, sem):
        wid = lax.axis_index("s") * NC + lax.axis_index("c")

        def run(idx_hbm, table_hbm, out_hbm, per_worker):
            base = wid * per_worker

            def body(c, _):
                off = base + c * CH
                pltpu.sync_copy(idx_hbm.at[pl.ds(off, CH)], idx_v)
                pltpu.async_copy(table_hbm.at[idx_v], rows_v, sem).wait()
                pltpu.sync_copy(rows_v, out_hbm.at[pl.ds(off, CH)])
                return 0

            lax.fori_loop(0, per_worker // CH, body, 0)

        run(idx1, t1, o1, P1)
        run(idx2, t2, o2, P2)
        run(idx3, t3, o3, P3)

    return k


_MB = B
_KB = 256
_NK1, _NK2, _NK3 = (L1 * H) // _KB, (L2 * H) // _KB, (L3 * H) // _KB
_NK = _NK1 + _NK2 + _NK3


def _tc_body(g1, g2, g3, w1, w2, w3, bi, wf, bf, out_ref, acc_ref):
    k = pl.program_id(0)

    @pl.when(k == 0)
    def _():
        acc_ref[...] = jnp.broadcast_to(bi[...], (_MB, NHID))

    @pl.when(k < _NK1)
    def _():
        acc_ref[...] += jnp.dot(g1[...].astype(jnp.bfloat16), w1[...],
                                preferred_element_type=jnp.float32)

    @pl.when((k >= _NK1) & (k < _NK1 + _NK2))
    def _():
        acc_ref[...] += jnp.dot(g2[...].astype(jnp.bfloat16), w2[...],
                                preferred_element_type=jnp.float32)

    @pl.when(k >= _NK1 + _NK2)
    def _():
        acc_ref[...] += jnp.dot(g3[...].astype(jnp.bfloat16), w3[...],
                                preferred_element_type=jnp.float32)

    @pl.when(k == _NK - 1)
    def _():
        r = acc_ref[...] * wf[...]
        out_ref[...] = jnp.sum(r, axis=1, keepdims=True) + bf[0, 0]


def _tc_mlp(g1, g2, g3, w1, w2, w3, b_inter, w_final, b_final):
    grid = (_NK,)

    def seg_spec(lo, nk):
        return pl.BlockSpec(
            (_MB, _KB),
            lambda k: (0, jnp.clip(k - lo, 0, nk - 1)),
        )

    def w_spec(lo, nk):
        return pl.BlockSpec(
            (_KB, NHID),
            lambda k: (jnp.clip(k - lo, 0, nk - 1), 0),
        )

    return pl.pallas_call(
        _tc_body,
        grid=grid,
        in_specs=[
            seg_spec(0, _NK1),
            seg_spec(_NK1, _NK2),
            seg_spec(_NK1 + _NK2, _NK3),
            w_spec(0, _NK1),
            w_spec(_NK1, _NK2),
            w_spec(_NK1 + _NK2, _NK3),
            pl.BlockSpec((1, NHID), lambda k: (0, 0)),
            pl.BlockSpec((1, NHID), lambda k: (0, 0)),
            pl.BlockSpec(memory_space=pltpu.SMEM),
        ],
        out_specs=pl.BlockSpec((_MB, 1), lambda k: (0, 0)),
        out_shape=jax.ShapeDtypeStruct((B, 1), jnp.float32),
        scratch_shapes=[pltpu.VMEM((_MB, NHID), jnp.float32)],
    )(g1, g2, g3, w1, w2, w3, b_inter, w_final, b_final)


def _rearrange_w(w_seg, lt):
    return w_seg.reshape(NHID, H, lt).transpose(2, 1, 0).reshape(lt * H, NHID)


def kernel(input1, input2, input3, title_emb, full_emb, cat_emb,
           W_inter, b_inter, W_final, b_final):
    idx1 = input1.reshape(-1).astype(jnp.int32)
    idx2 = input2.reshape(-1).astype(jnp.int32)
    idx3 = input3.reshape(-1).astype(jnp.int32)

    g1, g2, g3 = _sc_gather()(idx1, idx2, idx3, title_emb, full_emb, cat_emb)

    wb = W_inter.astype(jnp.bfloat16)
    w1 = _rearrange_w(wb[:, : H * L1], L1)
    w2 = _rearrange_w(wb[:, H * L1 : H * (L1 + L2)], L2)
    w3 = _rearrange_w(wb[:, H * (L1 + L2) :], L3)

    return _tc_mlp(
        g1.reshape(B, L1 * H), g2.reshape(B, L2 * H), g3.reshape(B, L3 * H),
        w1, w2, w3,
        b_inter.reshape(1, NHID),
        W_final.reshape(1, NHID),
        b_final.reshape(1, 1).astype(jnp.float32),
    )


# speedup vs baseline: 44.7304x; 1.2125x over previous
import functools

import jax
import jax.numpy as jnp
from jax import lax
from jax.experimental import pallas as pl
from jax.experimental.pallas import tpu as pltpu
from jax.experimental.pallas import tpu_sc as plsc

B = 4096
L1, L2, L3 = 20, 200, 26
H = 128
HW = H // 2
NHID = 256

NC, NS = 2, 16
NW = NC * NS
CH = 128

N1, N2, N3 = B * L1, B * L2, B * L3
P1, P2, P3 = N1 // NW, N2 // NW, N3 // NW


NBUF = 4


def _sc_gather():
    mesh = plsc.VectorSubcoreMesh(core_axis_name="c", subcore_axis_name="s")

    @functools.partial(
        pl.kernel,
        mesh=mesh,
        out_type=(
            jax.ShapeDtypeStruct((N1, H), jnp.float32),
            jax.ShapeDtypeStruct((N2, H), jnp.float32),
            jax.ShapeDtypeStruct((N3, H), jnp.float32),
        ),
        scratch_types=[
            pltpu.VMEM((P1,), jnp.int32),
            pltpu.VMEM((P2,), jnp.int32),
            pltpu.VMEM((P3,), jnp.int32),
            pltpu.VMEM((NBUF, CH, H), jnp.float32),
            pltpu.SemaphoreType.DMA((NBUF,)),
            pltpu.SemaphoreType.DMA((NBUF,)),
        ],
    )
    def k(idx1, idx2, idx3, t1, t2, t3, o1, o2, o3,
          idx1_v, idx2_v, idx3_v, rows_v, gsem, wsem):
        wid = lax.axis_index("s") * NC + lax.axis_index("c")

        def run(idx_hbm, idx_v, table_hbm, out_hbm, per_worker):
            n = per_worker // CH
            base = wid * per_worker
            pltpu.sync_copy(idx_hbm.at[pl.ds(base, per_worker)], idx_v)

            def gth(c, b):
                return pltpu.make_async_copy(
                    table_hbm.at[idx_v.at[pl.ds(c * CH, CH)]],
                    rows_v.at[b], gsem.at[b])

            def wb(c, b):
                return pltpu.make_async_copy(
                    rows_v.at[b], out_hbm.at[pl.ds(base + c * CH, CH)],
                    wsem.at[b])

            for b in range(NBUF):
                gth(b, b).start()

            m4 = ((n - NBUF) // NBUF) * NBUF

            def body(i, _):
                for b in range(NBUF):
                    c = i * NBUF + b
                    gth(c, b).wait()
                    wb(c, b).start()
                    wb(c, b).wait()
                    gth(c + NBUF, b).start()
                return 0

            lax.fori_loop(0, m4 // NBUF, body, 0)

            for cc in range(m4, n):
                b = cc % NBUF
                gth(cc, b).wait()
                wb(cc, b).start()
                wb(cc, b).wait()
                if cc + NBUF < n:
                    gth(cc + NBUF, b).start()

        run(idx1, idx1_v, t1, o1, P1)
        run(idx2, idx2_v, t2, o2, P2)
        run(idx3, idx3_v, t3, o3, P3)

    return k


_MB = B
_KB = 256
_NK1, _NK2, _NK3 = (L1 * H) // _KB, (L2 * H) // _KB, (L3 * H) // _KB
_NK = _NK1 + _NK2 + _NK3


def _tc_body(g1, g2, g3, w1, w2, w3, bi, wf, bf, out_ref, acc_ref):
    k = pl.program_id(0)

    @pl.when(k == 0)
    def _():
        acc_ref[...] = jnp.broadcast_to(bi[...], (_MB, NHID))

    @pl.when(k < _NK1)
    def _():
        acc_ref[...] += jnp.dot(g1[...].astype(jnp.bfloat16), w1[...],
                                preferred_element_type=jnp.float32)

    @pl.when((k >= _NK1) & (k < _NK1 + _NK2))
    def _():
        acc_ref[...] += jnp.dot(g2[...].astype(jnp.bfloat16), w2[...],
                                preferred_element_type=jnp.float32)

    @pl.when(k >= _NK1 + _NK2)
    def _():
        acc_ref[...] += jnp.dot(g3[...].astype(jnp.bfloat16), w3[...],
                                preferred_element_type=jnp.float32)

    @pl.when(k == _NK - 1)
    def _():
        r = acc_ref[...] * wf[...]
        out_ref[...] = jnp.sum(r, axis=1, keepdims=True) + bf[0, 0]


def _tc_mlp(g1, g2, g3, w1, w2, w3, b_inter, w_final, b_final):
    grid = (_NK,)

    def seg_spec(lo, nk):
        return pl.BlockSpec(
            (_MB, _KB),
            lambda k: (0, jnp.clip(k - lo, 0, nk - 1)),
        )

    def w_spec(lo, nk):
        return pl.BlockSpec(
            (_KB, NHID),
            lambda k: (jnp.clip(k - lo, 0, nk - 1), 0),
        )

    return pl.pallas_call(
        _tc_body,
        grid=grid,
        in_specs=[
            seg_spec(0, _NK1),
            seg_spec(_NK1, _NK2),
            seg_spec(_NK1 + _NK2, _NK3),
            w_spec(0, _NK1),
            w_spec(_NK1, _NK2),
            w_spec(_NK1 + _NK2, _NK3),
            pl.BlockSpec((1, NHID), lambda k: (0, 0)),
            pl.BlockSpec((1, NHID), lambda k: (0, 0)),
            pl.BlockSpec(memory_space=pltpu.SMEM),
        ],
        out_specs=pl.BlockSpec((_MB, 1), lambda k: (0, 0)),
        out_shape=jax.ShapeDtypeStruct((B, 1), jnp.float32),
        scratch_shapes=[pltpu.VMEM((_MB, NHID), jnp.float32)],
    )(g1, g2, g3, w1, w2, w3, b_inter, w_final, b_final)


def _rearrange_w(w_seg, lt):
    return w_seg.reshape(NHID, H, lt).transpose(2, 1, 0).reshape(lt * H, NHID)


def kernel(input1, input2, input3, title_emb, full_emb, cat_emb,
           W_inter, b_inter, W_final, b_final):
    idx1 = input1.reshape(-1).astype(jnp.int32)
    idx2 = input2.reshape(-1).astype(jnp.int32)
    idx3 = input3.reshape(-1).astype(jnp.int32)

    g1, g2, g3 = _sc_gather()(idx1, idx2, idx3, title_emb, full_emb, cat_emb)

    wb = W_inter.astype(jnp.bfloat16)
    w1 = _rearrange_w(wb[:, : H * L1], L1)
    w2 = _rearrange_w(wb[:, H * L1 : H * (L1 + L2)], L2)
    w3 = _rearrange_w(wb[:, H * (L1 + L2) :], L3)

    return _tc_mlp(
        g1.reshape(B, L1 * H), g2.reshape(B, L2 * H), g3.reshape(B, L3 * H),
        w1, w2, w3,
        b_inter.reshape(1, NHID),
        W_final.reshape(1, NHID),
        b_final.reshape(1, 1).astype(jnp.float32),
    )
